# Initial kernel scaffold; baseline (speedup 1.0000x reference)
import jax
import jax.numpy as jnp
from jax.experimental import pallas as pl


def kernel(x, pe):
    raise NotImplementedError("write your pallas kernel here")


# speedup vs baseline: 2.8667x; 2.8667x over previous
import jax
import jax.numpy as jnp
from jax.experimental import pallas as pl


_BLK_S = 512


def _pe_add_kernel(x_ref, pe_ref, o_ref):
    o_ref[...] = x_ref[...] + pe_ref[...]


def kernel(x, pe):
    batch, seq_len, d_model = x.shape
    num_s = seq_len // _BLK_S
    return pl.pallas_call(
        _pe_add_kernel,
        grid=(num_s, batch),
        in_specs=[
            pl.BlockSpec((1, _BLK_S, d_model), lambda s, b: (b, s, 0)),
            pl.BlockSpec((_BLK_S, d_model), lambda s, b: (s, 0)),
        ],
        out_specs=pl.BlockSpec((1, _BLK_S, d_model), lambda s, b: (b, s, 0)),
        out_shape=jax.ShapeDtypeStruct(x.shape, x.dtype),
    )(x, pe)


# speedup vs baseline: 3.1972x; 1.1153x over previous
import jax
import jax.numpy as jnp
from jax.experimental import pallas as pl


_BLK_S = 1024


def _pe_add_kernel(x_ref, pe_ref, o_ref):
    o_ref[...] = x_ref[...] + pe_ref[...]


def kernel(x, pe):
    batch, seq_len, d_model = x.shape
    num_s = seq_len // _BLK_S
    return pl.pallas_call(
        _pe_add_kernel,
        grid=(num_s, batch),
        in_specs=[
            pl.BlockSpec((1, _BLK_S, d_model), lambda s, b: (b, s, 0)),
            pl.BlockSpec((_BLK_S, d_model), lambda s, b: (s, 0)),
        ],
        out_specs=pl.BlockSpec((1, _BLK_S, d_model), lambda s, b: (b, s, 0)),
        out_shape=jax.ShapeDtypeStruct(x.shape, x.dtype),
    )(x, pe)
